# Initial kernel scaffold; baseline (speedup 1.0000x reference)
import jax
import jax.numpy as jnp
from jax.experimental import pallas as pl


def kernel(x, table):
    raise NotImplementedError("write your pallas kernel here")


# speedup vs baseline: 2.5640x; 2.5640x over previous
import jax
import jax.numpy as jnp
from jax import lax
from jax.experimental import pallas as pl
from jax.experimental.pallas import tpu as pltpu
from jax.experimental.pallas import tpu_sc as plsc

B = 16384
L = 50
D = 64
NC = 2
NS = 16
NW = NC * NS
RPW = B // NW
CB = 2
CIDX = CB * L
NCHUNK = RPW // CB
NBUF = 2
NV = D // 16


def _body(idx_hbm, table_hbm, out_hbm, idx_v, buf0, buf1, out_v, sem0, sem1):
    wid = lax.axis_index("s") * NC + lax.axis_index("c")
    pltpu.sync_copy(idx_hbm.at[wid], idx_v)
    bufs = (buf0, buf1)
    sems = (sem0, sem1)

    for b in range(NBUF):
        pltpu.async_copy(table_hbm.at[idx_v.at[b]], bufs[b], sems[b])

    inv = jnp.float32(1.0 / L)

    def reduce_chunk(c, src):
        for r in range(CB):
            accs = [src[r * L, pl.ds(d * 16, 16)] for d in range(NV)]
            for j in range(1, L):
                for d in range(NV):
                    accs[d] = accs[d] + src[r * L + j, pl.ds(d * 16, 16)]
            row = c * CB + r
            for d in range(NV):
                out_v[row, pl.ds(d * 16, 16)] = accs[d] * inv

    @pl.loop(0, NCHUNK // NBUF)
    def _chunks(c0):
        for b in range(NBUF):
            c = c0 * NBUF + b
            pltpu.make_async_copy(
                table_hbm.at[idx_v.at[c]], bufs[b], sems[b]).wait()
            reduce_chunk(c, bufs[b])
            nxt = c + NBUF

            @pl.when(nxt < NCHUNK)
            def _():
                pltpu.async_copy(table_hbm.at[idx_v.at[nxt]], bufs[b], sems[b])

    pltpu.sync_copy(out_v, out_hbm.at[pl.ds(wid * RPW, RPW)])


def kernel(x, table):
    idx = x.astype(jnp.int32).reshape(NW, NCHUNK, CIDX)
    mesh = plsc.VectorSubcoreMesh(
        core_axis_name="c", subcore_axis_name="s",
        num_cores=NC, num_subcores=NS)
    k = pl.kernel(
        _body,
        out_type=jax.ShapeDtypeStruct((B, D), jnp.float32),
        mesh=mesh,
        scratch_types=[
            pltpu.VMEM((NCHUNK, CIDX), jnp.int32),
            pltpu.VMEM((CIDX, D), jnp.float32),
            pltpu.VMEM((CIDX, D), jnp.float32),
            pltpu.VMEM((RPW, D), jnp.float32),
            pltpu.SemaphoreType.DMA,
            pltpu.SemaphoreType.DMA,
        ],
        compiler_params=pltpu.CompilerParams(use_tc_tiling_on_sc=False),
    )
    return k(idx, table)
